# baseline (speedup 1.0000x reference)
import functools

import jax
import jax.numpy as jnp
from jax import lax
from jax.experimental import pallas as pl
from jax.experimental.pallas import tpu as pltpu
from jax.experimental.pallas import tpu_sc as plsc

_B, _N, _C = 8, 115200, 3
_TOTAL = _B * _N * _C
_L = 16
_ROWS = _TOTAL // _L
_NC, _NS = 2, 16
_NW = _NC * _NS
_ROWS_W = _ROWS // _NW
_CH_ROWS = 1800
_NCHUNK = _ROWS_W // _CH_ROWS

_mesh = plsc.VectorSubcoreMesh(core_axis_name="c", subcore_axis_name="s")


@functools.partial(
    pl.kernel,
    mesh=_mesh,
    out_type=jax.ShapeDtypeStruct((_ROWS, _L), jnp.float32),
    scratch_types=[
        pltpu.VMEM((_CH_ROWS, _L), jnp.float32),
        pltpu.VMEM((_CH_ROWS, _L), jnp.float32),
    ],
    compiler_params=pltpu.CompilerParams(use_tc_tiling_on_sc=False),
)
def _sc_add(x_hbm, f_hbm, o_hbm, a_v, b_v):
    wid = lax.axis_index("s") * _NC + lax.axis_index("c")
    base = wid * _ROWS_W

    def chunk(ci, carry):
        off = base + ci * _CH_ROWS
        pltpu.sync_copy(x_hbm.at[pl.ds(off, _CH_ROWS)], a_v)
        pltpu.sync_copy(f_hbm.at[pl.ds(off, _CH_ROWS)], b_v)

        def body(i, c2):
            a_v[i] = a_v[i] + b_v[i]
            return c2

        lax.fori_loop(0, _CH_ROWS, body, 0, unroll=8)
        pltpu.sync_copy(a_v, o_hbm.at[pl.ds(off, _CH_ROWS)])
        return carry

    lax.fori_loop(0, _NCHUNK, chunk, 0)


def kernel(xyz1, upsampled_flow):
    x = xyz1.reshape(_ROWS, _L)
    f = upsampled_flow.reshape(_ROWS, _L)
    out = _sc_add(x, f)
    return out.reshape(_B, _N, _C)


# speedup vs baseline: 89.5943x; 89.5943x over previous
import functools

import jax
import jax.numpy as jnp
from jax import lax
from jax.experimental import pallas as pl
from jax.experimental.pallas import tpu as pltpu
from jax.experimental.pallas import tpu_sc as plsc

_B, _N, _C = 8, 115200, 3
_ROWS = _C * _B
_COLS = _N
_NC, _NS = 2, 16
_NW = _NC * _NS
_CW = 1152
_CPR = _COLS // _CW
_NCHUNK = (_ROWS // 8) * _CPR
_K_MAX = (_NCHUNK + _NW - 1) // _NW

_mesh = plsc.VectorSubcoreMesh(core_axis_name="c", subcore_axis_name="s")


@functools.partial(
    pl.kernel,
    mesh=_mesh,
    out_type=jax.ShapeDtypeStruct((_ROWS, _COLS), jnp.float32),
    scratch_types=[
        pltpu.VMEM((8, _CW), jnp.float32),
        pltpu.VMEM((8, _CW), jnp.float32),
    ],
)
def _sc_add(x_hbm, f_hbm, o_hbm, a_v, b_v):
    wid = lax.axis_index("s") * _NC + lax.axis_index("c")

    def step(k, carry):
        c = wid + k * _NW

        @pl.when(c < _NCHUNK)
        def _():
            r0 = (c // _CPR) * 8
            c0 = (c % _CPR) * _CW
            src = (pl.ds(r0, 8), pl.ds(c0, _CW))
            pltpu.sync_copy(x_hbm.at[src], a_v)
            pltpu.sync_copy(f_hbm.at[src], b_v)

            def row(r, c2):
                for cc in range(_CW // 16):
                    sl = pl.ds(cc * 16, 16)
                    a_v[r, sl] = a_v[r, sl] + b_v[r, sl]
                return c2

            lax.fori_loop(0, 8, row, 0)
            pltpu.sync_copy(a_v, o_hbm.at[src])

        return carry

    lax.fori_loop(0, _K_MAX, step, 0)


def kernel(xyz1, upsampled_flow):
    x = jnp.transpose(xyz1, (2, 0, 1)).reshape(_ROWS, _COLS)
    f = jnp.transpose(upsampled_flow, (2, 0, 1)).reshape(_ROWS, _COLS)
    out = _sc_add(x, f)
    return jnp.transpose(out.reshape(_C, _B, _N), (1, 2, 0))


# speedup vs baseline: 143.0126x; 1.5962x over previous
import functools

import jax
import jax.numpy as jnp
from jax import lax
from jax.experimental import pallas as pl
from jax.experimental.pallas import tpu as pltpu
from jax.experimental.pallas import tpu_sc as plsc

_B, _N, _C = 8, 115200, 3
_ROWS = _C * _B
_COLS = _N
_NC, _NS = 2, 16
_NW = _NC * _NS
_CW = 1152
_CPR = _COLS // _CW
_NCHUNK = (_ROWS // 8) * _CPR
_K_MAX = (_NCHUNK + _NW - 1) // _NW

_mesh = plsc.VectorSubcoreMesh(core_axis_name="c", subcore_axis_name="s")


@functools.partial(
    pl.kernel,
    mesh=_mesh,
    out_type=jax.ShapeDtypeStruct((_ROWS, _COLS), jnp.float32),
    scratch_types=[
        pltpu.VMEM((8, _CW), jnp.float32),
        pltpu.VMEM((8, _CW), jnp.float32),
        pltpu.VMEM((8, _CW), jnp.float32),
        pltpu.VMEM((8, _CW), jnp.float32),
        pltpu.VMEM((8, _CW), jnp.float32),
        pltpu.VMEM((8, _CW), jnp.float32),
        pltpu.SemaphoreType.DMA,
        pltpu.SemaphoreType.DMA,
        pltpu.SemaphoreType.DMA,
        pltpu.SemaphoreType.DMA,
    ],
)
def _sc_add(x_hbm, f_hbm, o_hbm, a0, b0, a1, b1, o0, o1, si0, si1, so0, so1):
    wid = lax.axis_index("s") * _NC + lax.axis_index("c")
    av = (a0, a1)
    bv = (b0, b1)
    ov = (o0, o1)
    si = (si0, si1)
    so = (so0, so1)

    def src_slice(k):
        c = wid + k * _NW
        return (pl.ds((c // _CPR) * 8, 8), pl.ds((c % _CPR) * _CW, _CW))

    def start_in(k, p):
        @pl.when(wid + k * _NW < _NCHUNK)
        def _():
            s = src_slice(k)
            pltpu.async_copy(x_hbm.at[s], av[p], si[p])
            pltpu.async_copy(f_hbm.at[s], bv[p], si[p])

    start_in(0, 0)
    start_in(1, 1)

    @pl.loop(0, _K_MAX, step=2)
    def _(kk):
        for p in range(2):
            k = kk + p

            @pl.when(wid + k * _NW < _NCHUNK)
            def _():
                s = src_slice(k)
                pltpu.make_async_copy(x_hbm.at[s], av[p], si[p]).wait()
                pltpu.make_async_copy(f_hbm.at[s], bv[p], si[p]).wait()

                @pl.when(k >= 2)
                def _():
                    pltpu.make_async_copy(ov[p], o_hbm.at[s], so[p]).wait()

                def row(r, c2):
                    for cc in range(_CW // 16):
                        sl = pl.ds(cc * 16, 16)
                        ov[p][r, sl] = av[p][r, sl] + bv[p][r, sl]
                    return c2

                lax.fori_loop(0, 8, row, 0)
                pltpu.async_copy(ov[p], o_hbm.at[s], so[p])
                start_in(k + 2, p)

    drain = (pl.ds(0, 8), pl.ds(0, _CW))
    for p in range(2):
        pltpu.make_async_copy(ov[p], o_hbm.at[drain], so[p]).wait()


def kernel(xyz1, upsampled_flow):
    x = jnp.transpose(xyz1, (2, 0, 1)).reshape(_ROWS, _COLS)
    f = jnp.transpose(upsampled_flow, (2, 0, 1)).reshape(_ROWS, _COLS)
    out = _sc_add(x, f)
    return jnp.transpose(out.reshape(_C, _B, _N), (1, 2, 0))


# speedup vs baseline: 143.2848x; 1.0019x over previous
import functools

import jax
import jax.numpy as jnp
from jax import lax
from jax.experimental import pallas as pl
from jax.experimental.pallas import tpu as pltpu
from jax.experimental.pallas import tpu_sc as plsc

_B, _N, _C = 8, 115200, 3
_ROWS = _C * _B
_COLS = _N
_NC, _NS = 2, 16
_NW = _NC * _NS
_CW = 1152
_CPR = _COLS // _CW
_NCHUNK = (_ROWS // 8) * _CPR
_K_MAX = (_NCHUNK + _NW - 1) // _NW

_mesh = plsc.VectorSubcoreMesh(core_axis_name="c", subcore_axis_name="s")


@functools.partial(
    pl.kernel,
    mesh=_mesh,
    out_type=jax.ShapeDtypeStruct((_ROWS, _COLS), jnp.float32),
    scratch_types=[
        pltpu.VMEM((8, _CW), jnp.float32),
        pltpu.VMEM((8, _CW), jnp.float32),
        pltpu.VMEM((8, _CW), jnp.float32),
        pltpu.VMEM((8, _CW), jnp.float32),
        pltpu.VMEM((8, _CW), jnp.float32),
        pltpu.VMEM((8, _CW), jnp.float32),
        pltpu.SemaphoreType.DMA,
        pltpu.SemaphoreType.DMA,
        pltpu.SemaphoreType.DMA,
        pltpu.SemaphoreType.DMA,
    ],
    compiler_params=pltpu.CompilerParams(skip_device_barrier=True),
)
def _sc_add(x_hbm, f_hbm, o_hbm, a0, b0, a1, b1, o0, o1, si0, si1, so0, so1):
    wid = lax.axis_index("s") * _NC + lax.axis_index("c")
    av = (a0, a1)
    bv = (b0, b1)
    ov = (o0, o1)
    si = (si0, si1)
    so = (so0, so1)

    def src_slice(k):
        c = wid + k * _NW
        return (pl.ds((c // _CPR) * 8, 8), pl.ds((c % _CPR) * _CW, _CW))

    def start_in(k, p):
        @pl.when(wid + k * _NW < _NCHUNK)
        def _():
            s = src_slice(k)
            pltpu.async_copy(x_hbm.at[s], av[p], si[p])
            pltpu.async_copy(f_hbm.at[s], bv[p], si[p])

    start_in(0, 0)
    start_in(1, 1)

    @pl.loop(0, _K_MAX, step=2)
    def _(kk):
        for p in range(2):
            k = kk + p

            @pl.when(wid + k * _NW < _NCHUNK)
            def _():
                s = src_slice(k)
                pltpu.make_async_copy(x_hbm.at[s], av[p], si[p]).wait()
                pltpu.make_async_copy(f_hbm.at[s], bv[p], si[p]).wait()

                @pl.when(k >= 2)
                def _():
                    pltpu.make_async_copy(ov[p], o_hbm.at[s], so[p]).wait()

                def row(r, c2):
                    for cc in range(_CW // 16):
                        sl = pl.ds(cc * 16, 16)
                        ov[p][r, sl] = av[p][r, sl] + bv[p][r, sl]
                    return c2

                lax.fori_loop(0, 8, row, 0)
                pltpu.async_copy(ov[p], o_hbm.at[s], so[p])
                start_in(k + 2, p)

    drain = (pl.ds(0, 8), pl.ds(0, _CW))
    for p in range(2):
        pltpu.make_async_copy(ov[p], o_hbm.at[drain], so[p]).wait()


def kernel(xyz1, upsampled_flow):
    x = jnp.transpose(xyz1, (2, 0, 1)).reshape(_ROWS, _COLS)
    f = jnp.transpose(upsampled_flow, (2, 0, 1)).reshape(_ROWS, _COLS)
    out = _sc_add(x, f)
    return jnp.transpose(out.reshape(_C, _B, _N), (1, 2, 0))
